# baseline (speedup 1.0000x reference)
import jax
import jax.numpy as jnp
from jax import lax
from jax.experimental import pallas as pl
from jax.experimental.pallas import tpu as pltpu
from jax.experimental.pallas import tpu_sc as plsc

N = 10000
E = 320000
D = 128
DE = 16

NC = 2
NS = 16
NW = NC * NS
EPW = E // NW

GCB = 80
NGC = EPW // GCB

SCB = 80
NSC = EPW // SCB

NPAD = 10240
RPS = NPAD // NS

_PB = 400


def _prep_body(h_ref, w0t_ref, b0_ref, wd_ref, bd_ref, wsrc_ref,
               out_ref, td_ref, ts_ref):
    hb = h_ref[...]
    ob = jax.nn.softplus(
        jnp.dot(hb, w0t_ref[...], preferred_element_type=jnp.float32)
        + b0_ref[...])
    out_ref[...] = ob
    td_ref[...] = (jnp.dot(ob, wd_ref[...], preferred_element_type=jnp.float32)
                   + bd_ref[...]).astype(jnp.bfloat16)
    ts_ref[...] = jnp.dot(
        ob, wsrc_ref[...],
        preferred_element_type=jnp.float32).astype(jnp.bfloat16)


_GNB = 3


def _gather_body(td_hbm, ts_hbm, dst3_hbm, src3_hbm, gd_hbm, gs_hbm,
                 idxd2, idxs2, rds, rss, gsems, wsems):
    c = lax.axis_index("c")
    s = lax.axis_index("s")
    wid = c * NS + s
    ebase = wid * EPW

    pltpu.sync_copy(dst3_hbm.at[wid], idxd2)
    pltpu.sync_copy(src3_hbm.at[wid], idxs2)

    def issue(k, b):
        dd = pltpu.async_copy(td_hbm.at[idxd2.at[k]], rds[b], gsems[b])
        ds = pltpu.async_copy(ts_hbm.at[idxs2.at[k]], rss[b], gsems[b])
        return dd, ds

    def drain_write(b):
        pltpu.make_async_copy(rds[b], gd_hbm.at[pl.ds(0, GCB)], wsems[b]).wait()
        pltpu.make_async_copy(rss[b], gs_hbm.at[pl.ds(0, GCB)], wsems[b]).wait()

    def write(k, b):
        pltpu.async_copy(rds[b], gd_hbm.at[pl.ds(ebase + k * GCB, GCB)],
                         wsems[b])
        pltpu.async_copy(rss[b], gs_hbm.at[pl.ds(ebase + k * GCB, GCB)],
                         wsems[b])

    def body(i, carry):
        k0 = i * _GNB

        @pl.when(i > 0)
        def _():
            for b in range(_GNB):
                drain_write(b)

        descs = [issue(k0 + b, b) for b in range(_GNB)]
        for b in range(_GNB):
            descs[b][0].wait()
            descs[b][1].wait()
            write(k0 + b, b)
        return carry

    lax.fori_loop(0, NGC // _GNB, body, 0)
    for b in range(NGC % _GNB):
        k = (NGC // _GNB) * _GNB + b
        drain_write(b)
        da, db = issue(k, b)
        da.wait()
        db.wait()
        write(k, b)
    for b in range(_GNB):
        if b >= NGC % _GNB:
            drain_write(b)
    for b in range(NGC % _GNB):
        drain_write(b)


_MB = 2000


def _math_body(gd_ref, gs_ref, ea_ref, w2_ref, w16_ref, sb_ref, m_ref):
    z0 = (gd_ref[...].astype(jnp.float32)
          + gs_ref[...].astype(jnp.float32))
    eav = jax.nn.softplus(
        jnp.sum(ea_ref[...] * w16_ref[...], axis=1, keepdims=True)
        + sb_ref[0, 0])
    z = z0 + eav * w2_ref[...]
    m_ref[...] = jax.nn.sigmoid(z[:, :D]) * jax.nn.softplus(z[:, D:])


def _scatter_body(m_hbm, dst_hbm, z_hbm, part_hbm,
                  idx0, idx1, mb0, mb1, sem0, sem1, agg_sh):
    c = lax.axis_index("c")
    s = lax.axis_index("s")
    wid = c * NS + s
    row0 = s * RPS
    ebase = wid * EPW

    pltpu.sync_copy(z_hbm.at[pl.ds(row0, RPS)], agg_sh.at[pl.ds(row0, RPS)])
    plsc.subcore_barrier()

    def load(k, idx, mb, sem):
        base = ebase + k * SCB
        pltpu.sync_copy(dst_hbm.at[pl.ds(base, SCB)], idx)
        return pltpu.async_copy(m_hbm.at[pl.ds(base, SCB)], mb, sem)

    def body(i, carry):
        k0 = i * 2
        k1 = k0 + 1
        d0 = load(k0, idx0, mb0, sem0)
        d1 = load(k1, idx1, mb1, sem1)
        d0.wait()
        pltpu.sync_copy(mb0, agg_sh.at[idx0], add=True)
        d1.wait()
        pltpu.sync_copy(mb1, agg_sh.at[idx1], add=True)
        return carry

    lax.fori_loop(0, NSC // 2, body, 0)
    load(NSC - 1, idx0, mb0, sem0).wait()
    pltpu.sync_copy(mb0, agg_sh.at[idx0], add=True)

    plsc.subcore_barrier()
    pltpu.sync_copy(agg_sh.at[pl.ds(row0, RPS)],
                    part_hbm.at[c, pl.ds(row0, RPS)])


def _fin_body(p0_ref, p1_ref, out_ref_in, g_ref, b_ref, y_ref):
    agg = p0_ref[...] + p1_ref[...]
    mean = jnp.mean(agg, axis=0, keepdims=True)
    var = jnp.mean((agg - mean) ** 2, axis=0, keepdims=True)
    bn = (agg - mean) * lax.rsqrt(var + 1e-5) * g_ref[...] + b_ref[...]
    y_ref[...] = 2.0 * out_ref_in[...] + bn


def kernel(h, edge_index, edge_weight, edge_attr, data, lin0_W, lin0_b,
           short_W, short_b, Wf, bf, Ws, bs, bn_gamma, bn_beta):
    w0t = lin0_W.T
    wd = jnp.concatenate([Wf[:, :D].T, Ws[:, :D].T], axis=1)
    bd = jnp.concatenate([bf, bs])[None, :]
    wsrc = jnp.concatenate([Wf[:, D:2 * D].T, Ws[:, D:2 * D].T], axis=1)
    w2 = jnp.concatenate([Wf[:, 2 * D], Ws[:, 2 * D]])[None, :]
    src = edge_index[0]
    dst = edge_index[1]
    b0 = lin0_b[None, :]

    out, td, ts = pl.pallas_call(
        _prep_body,
        grid=(N // _PB,),
        in_specs=[
            pl.BlockSpec((_PB, D), lambda i: (i, 0)),
            pl.BlockSpec((D, D), lambda i: (0, 0)),
            pl.BlockSpec((1, D), lambda i: (0, 0)),
            pl.BlockSpec((D, 2 * D), lambda i: (0, 0)),
            pl.BlockSpec((1, 2 * D), lambda i: (0, 0)),
            pl.BlockSpec((D, 2 * D), lambda i: (0, 0)),
        ],
        out_specs=[
            pl.BlockSpec((_PB, D), lambda i: (i, 0)),
            pl.BlockSpec((_PB, 2 * D), lambda i: (i, 0)),
            pl.BlockSpec((_PB, 2 * D), lambda i: (i, 0)),
        ],
        out_shape=[
            jax.ShapeDtypeStruct((N, D), jnp.float32),
            jax.ShapeDtypeStruct((N, 2 * D), jnp.bfloat16),
            jax.ShapeDtypeStruct((N, 2 * D), jnp.bfloat16),
        ],
    )(h, w0t, b0, wd, bd, wsrc)

    td_i = lax.bitcast_convert_type(td.reshape(N, D, 2), jnp.int32)
    ts_i = lax.bitcast_convert_type(ts.reshape(N, D, 2), jnp.int32)
    dst3 = dst.reshape(NW, NGC, GCB)
    src3 = src.reshape(NW, NGC, GCB)
    gd, gs = pl.kernel(
        _gather_body,
        out_type=[jax.ShapeDtypeStruct((E, D), jnp.int32),
                  jax.ShapeDtypeStruct((E, D), jnp.int32)],
        mesh=plsc.VectorSubcoreMesh(core_axis_name="c", subcore_axis_name="s"),
        scratch_types=[
            pltpu.VMEM((NGC, GCB), jnp.int32),
            pltpu.VMEM((NGC, GCB), jnp.int32),
            [pltpu.VMEM((GCB, D), jnp.int32)] * _GNB,
            [pltpu.VMEM((GCB, D), jnp.int32)] * _GNB,
            [pltpu.SemaphoreType.DMA] * _GNB,
            [pltpu.SemaphoreType.DMA] * _GNB,
        ],
    )(td_i, ts_i, dst3, src3)

    gd_bf = lax.bitcast_convert_type(gd, jnp.bfloat16).reshape(E, 2 * D)
    gs_bf = lax.bitcast_convert_type(gs, jnp.bfloat16).reshape(E, 2 * D)
    m = pl.pallas_call(
        _math_body,
        grid=(E // _MB,),
        in_specs=[
            pl.BlockSpec((_MB, 2 * D), lambda i: (i, 0)),
            pl.BlockSpec((_MB, 2 * D), lambda i: (i, 0)),
            pl.BlockSpec((_MB, DE), lambda i: (i, 0)),
            pl.BlockSpec((1, 2 * D), lambda i: (0, 0)),
            pl.BlockSpec((1, DE), lambda i: (0, 0)),
            pl.BlockSpec((1, 1), lambda i: (0, 0)),
        ],
        out_specs=pl.BlockSpec((_MB, D), lambda i: (i, 0)),
        out_shape=jax.ShapeDtypeStruct((E, D), jnp.float32),
    )(gd_bf, gs_bf, edge_attr, w2, short_W, short_b[None, :])

    zeros = jnp.zeros((NPAD, D), jnp.float32)
    parts = pl.kernel(
        _scatter_body,
        out_type=jax.ShapeDtypeStruct((NC, NPAD, D), jnp.float32),
        mesh=plsc.VectorSubcoreMesh(core_axis_name="c", subcore_axis_name="s"),
        scratch_types=[
            pltpu.VMEM((SCB,), jnp.int32),
            pltpu.VMEM((SCB,), jnp.int32),
            pltpu.VMEM((SCB, D), jnp.float32),
            pltpu.VMEM((SCB, D), jnp.float32),
            pltpu.SemaphoreType.DMA,
            pltpu.SemaphoreType.DMA,
            pltpu.VMEM_SHARED((NPAD, D), jnp.float32),
        ],
    )(m, dst, zeros)

    y = pl.pallas_call(
        _fin_body,
        out_shape=jax.ShapeDtypeStruct((N, D), jnp.float32),
    )(parts[0, :N], parts[1, :N], out, bn_gamma[None, :], bn_beta[None, :])
    return y


# speedup vs baseline: 3.3952x; 3.3952x over previous
import jax
import jax.numpy as jnp
from jax import lax
from jax.experimental import pallas as pl
from jax.experimental.pallas import tpu as pltpu
from jax.experimental.pallas import tpu_sc as plsc

N = 10000
E = 320000
D = 128
DE = 16

NC = 2
NS = 16
NW = NC * NS
EPW = E // NW

GCB = 40
NGC = EPW // GCB

SCB = 80
NSC = EPW // SCB

NPAD = 10240
RPS = NPAD // NS

_PB = 400


def _prep_body(h_ref, w0t_ref, b0_ref, wd_ref, bd_ref, wsrc_ref,
               out_ref, td_ref, ts_ref):
    hb = h_ref[...]
    ob = jax.nn.softplus(
        jnp.dot(hb, w0t_ref[...], preferred_element_type=jnp.float32)
        + b0_ref[...])
    out_ref[...] = ob
    td_ref[...] = (jnp.dot(ob, wd_ref[...], preferred_element_type=jnp.float32)
                   + bd_ref[...])
    ts_ref[...] = jnp.dot(ob, wsrc_ref[...], preferred_element_type=jnp.float32)


_GNB = 3


def _gather_body(td_hbm, ts_hbm, dst3_hbm, src3_hbm, g_hbm,
                 idxd2, idxs2, rds, rss, gsems, wsems):
    c = lax.axis_index("c")
    s = lax.axis_index("s")
    wid = c * NS + s
    ebase = wid * EPW

    pltpu.sync_copy(dst3_hbm.at[wid], idxd2)
    pltpu.sync_copy(src3_hbm.at[wid], idxs2)

    def issue(k, b):
        dd = pltpu.async_copy(td_hbm.at[idxd2.at[k]], rds[b], gsems[b])
        ds = pltpu.async_copy(ts_hbm.at[idxs2.at[k]], rss[b], gsems[b])
        return dd, ds

    def add(b):
        rd, rs = rds[b], rss[b]

        def ab(e, carry):
            for gg in range(16):
                sl = pl.ds(gg * 16, 16)
                rd[e, sl] = rd[e, sl] + rs[e, sl]
            return carry
        lax.fori_loop(0, GCB, ab, 0)

    def drain_write(b):
        pltpu.make_async_copy(rds[b], g_hbm.at[pl.ds(0, GCB)], wsems[b]).wait()

    def body(i, carry):
        k0 = i * _GNB

        @pl.when(i > 0)
        def _():
            for b in range(_GNB):
                drain_write(b)

        descs = [issue(k0 + b, b) for b in range(_GNB)]
        for b in range(_GNB):
            descs[b][0].wait()
            descs[b][1].wait()
            add(b)
            pltpu.async_copy(
                rds[b], g_hbm.at[pl.ds(ebase + (k0 + b) * GCB, GCB)], wsems[b])
        return carry

    lax.fori_loop(0, NGC // _GNB, body, 0)
    for b in range(NGC % _GNB):
        k = (NGC // _GNB) * _GNB + b
        drain_write(b)
        da, db = issue(k, b)
        da.wait()
        db.wait()
        add(b)
        pltpu.async_copy(
            rds[b], g_hbm.at[pl.ds(ebase + k * GCB, GCB)], wsems[b])
    for b in range(_GNB):
        if b >= NGC % _GNB:
            drain_write(b)
    for b in range(NGC % _GNB):
        drain_write(b)


_MB = 2000


def _math_body(g_ref, ea_ref, w2_ref, w16_ref, sb_ref, m_ref):
    eav = jax.nn.softplus(
        jnp.sum(ea_ref[...] * w16_ref[...], axis=1, keepdims=True)
        + sb_ref[0, 0])
    z = g_ref[...] + eav * w2_ref[...]
    m_ref[...] = jax.nn.sigmoid(z[:, :D]) * jax.nn.softplus(z[:, D:])


def _scatter_body(m_hbm, dst_hbm, z_hbm, part_hbm,
                  idx0, idx1, mb0, mb1, sem0, sem1, agg_sh):
    c = lax.axis_index("c")
    s = lax.axis_index("s")
    wid = c * NS + s
    row0 = s * RPS
    ebase = wid * EPW

    pltpu.sync_copy(z_hbm.at[pl.ds(row0, RPS)], agg_sh.at[pl.ds(row0, RPS)])
    plsc.subcore_barrier()

    def load(k, idx, mb, sem):
        base = ebase + k * SCB
        pltpu.sync_copy(dst_hbm.at[pl.ds(base, SCB)], idx)
        return pltpu.async_copy(m_hbm.at[pl.ds(base, SCB)], mb, sem)

    def body(i, carry):
        k0 = i * 2
        k1 = k0 + 1
        d0 = load(k0, idx0, mb0, sem0)
        d1 = load(k1, idx1, mb1, sem1)
        d0.wait()
        pltpu.sync_copy(mb0, agg_sh.at[idx0], add=True)
        d1.wait()
        pltpu.sync_copy(mb1, agg_sh.at[idx1], add=True)
        return carry

    lax.fori_loop(0, NSC // 2, body, 0)
    load(NSC - 1, idx0, mb0, sem0).wait()
    pltpu.sync_copy(mb0, agg_sh.at[idx0], add=True)

    plsc.subcore_barrier()
    pltpu.sync_copy(agg_sh.at[pl.ds(row0, RPS)],
                    part_hbm.at[c, pl.ds(row0, RPS)])


def _fin_body(p0_ref, p1_ref, out_ref_in, g_ref, b_ref, y_ref):
    agg = p0_ref[...] + p1_ref[...]
    mean = jnp.mean(agg, axis=0, keepdims=True)
    var = jnp.mean((agg - mean) ** 2, axis=0, keepdims=True)
    bn = (agg - mean) * lax.rsqrt(var + 1e-5) * g_ref[...] + b_ref[...]
    y_ref[...] = 2.0 * out_ref_in[...] + bn


def kernel(h, edge_index, edge_weight, edge_attr, data, lin0_W, lin0_b,
           short_W, short_b, Wf, bf, Ws, bs, bn_gamma, bn_beta):
    w0t = lin0_W.T
    wd = jnp.concatenate([Wf[:, :D].T, Ws[:, :D].T], axis=1)
    bd = jnp.concatenate([bf, bs])[None, :]
    wsrc = jnp.concatenate([Wf[:, D:2 * D].T, Ws[:, D:2 * D].T], axis=1)
    w2 = jnp.concatenate([Wf[:, 2 * D], Ws[:, 2 * D]])[None, :]
    src = edge_index[0]
    dst = edge_index[1]
    b0 = lin0_b[None, :]

    out, td, ts = pl.pallas_call(
        _prep_body,
        grid=(N // _PB,),
        in_specs=[
            pl.BlockSpec((_PB, D), lambda i: (i, 0)),
            pl.BlockSpec((D, D), lambda i: (0, 0)),
            pl.BlockSpec((1, D), lambda i: (0, 0)),
            pl.BlockSpec((D, 2 * D), lambda i: (0, 0)),
            pl.BlockSpec((1, 2 * D), lambda i: (0, 0)),
            pl.BlockSpec((D, 2 * D), lambda i: (0, 0)),
        ],
        out_specs=[
            pl.BlockSpec((_PB, D), lambda i: (i, 0)),
            pl.BlockSpec((_PB, 2 * D), lambda i: (i, 0)),
            pl.BlockSpec((_PB, 2 * D), lambda i: (i, 0)),
        ],
        out_shape=[
            jax.ShapeDtypeStruct((N, D), jnp.float32),
            jax.ShapeDtypeStruct((N, 2 * D), jnp.float32),
            jax.ShapeDtypeStruct((N, 2 * D), jnp.float32),
        ],
    )(h, w0t, b0, wd, bd, wsrc)

    dst3 = dst.reshape(NW, NGC, GCB)
    src3 = src.reshape(NW, NGC, GCB)
    g = pl.kernel(
        _gather_body,
        out_type=jax.ShapeDtypeStruct((E, 2 * D), jnp.float32),
        mesh=plsc.VectorSubcoreMesh(core_axis_name="c", subcore_axis_name="s"),
        scratch_types=[
            pltpu.VMEM((NGC, GCB), jnp.int32),
            pltpu.VMEM((NGC, GCB), jnp.int32),
            [pltpu.VMEM((GCB, 2 * D), jnp.float32)] * _GNB,
            [pltpu.VMEM((GCB, 2 * D), jnp.float32)] * _GNB,
            [pltpu.SemaphoreType.DMA] * _GNB,
            [pltpu.SemaphoreType.DMA] * _GNB,
        ],
    )(td, ts, dst3, src3)

    m = pl.pallas_call(
        _math_body,
        grid=(E // _MB,),
        in_specs=[
            pl.BlockSpec((_MB, 2 * D), lambda i: (i, 0)),
            pl.BlockSpec((_MB, DE), lambda i: (i, 0)),
            pl.BlockSpec((1, 2 * D), lambda i: (0, 0)),
            pl.BlockSpec((1, DE), lambda i: (0, 0)),
            pl.BlockSpec((1, 1), lambda i: (0, 0)),
        ],
        out_specs=pl.BlockSpec((_MB, D), lambda i: (i, 0)),
        out_shape=jax.ShapeDtypeStruct((E, D), jnp.float32),
    )(g, edge_attr, w2, short_W, short_b[None, :])

    zeros = jnp.zeros((NPAD, D), jnp.float32)
    parts = pl.kernel(
        _scatter_body,
        out_type=jax.ShapeDtypeStruct((NC, NPAD, D), jnp.float32),
        mesh=plsc.VectorSubcoreMesh(core_axis_name="c", subcore_axis_name="s"),
        scratch_types=[
            pltpu.VMEM((SCB,), jnp.int32),
            pltpu.VMEM((SCB,), jnp.int32),
            pltpu.VMEM((SCB, D), jnp.float32),
            pltpu.VMEM((SCB, D), jnp.float32),
            pltpu.SemaphoreType.DMA,
            pltpu.SemaphoreType.DMA,
            pltpu.VMEM_SHARED((NPAD, D), jnp.float32),
        ],
    )(m, dst, zeros)

    y = pl.pallas_call(
        _fin_body,
        out_shape=jax.ShapeDtypeStruct((N, D), jnp.float32),
    )(parts[0, :N], parts[1, :N], out, bn_gamma[None, :], bn_beta[None, :])
    return y


# speedup vs baseline: 3.4239x; 1.0085x over previous
import jax
import jax.numpy as jnp
from jax import lax
from jax.experimental import pallas as pl
from jax.experimental.pallas import tpu as pltpu
from jax.experimental.pallas import tpu_sc as plsc

N = 10000
E = 320000
D = 128
DE = 16

NC = 2
NS = 16
NW = NC * NS
EPW = E // NW

GCB = 40
NGC = EPW // GCB

SCB = 80
NSC = EPW // SCB

NPAD = 10240
RPS = NPAD // NS

_PB = 400


def _prep_body(h_ref, w0t_ref, b0_ref, wd_ref, bd_ref, wsrc_ref,
               out_ref, td_ref, ts_ref):
    hb = h_ref[...]
    ob = jax.nn.softplus(
        jnp.dot(hb, w0t_ref[...], preferred_element_type=jnp.float32)
        + b0_ref[...])
    out_ref[...] = ob
    td_ref[...] = (jnp.dot(ob, wd_ref[...], preferred_element_type=jnp.float32)
                   + bd_ref[...])
    ts_ref[...] = jnp.dot(ob, wsrc_ref[...], preferred_element_type=jnp.float32)


_GNB = 3


def _gather_body(td_hbm, ts_hbm, dst3_hbm, src3_hbm, g_hbm,
                 idxd2, idxs2, rds, rss, gsems, wsems):
    c = lax.axis_index("c")
    s = lax.axis_index("s")
    wid = c * NS + s
    ebase = wid * EPW

    pltpu.sync_copy(dst3_hbm.at[wid], idxd2)
    pltpu.sync_copy(src3_hbm.at[wid], idxs2)

    def issue(k, b):
        dd = pltpu.async_copy(td_hbm.at[idxd2.at[k]], rds[b], gsems[b])
        ds = pltpu.async_copy(ts_hbm.at[idxs2.at[k]], rss[b], gsems[b])
        return dd, ds

    def add(b):
        rd, rs = rds[b], rss[b]

        def ab(e, carry):
            for gg in range(16):
                sl = pl.ds(gg * 16, 16)
                rd[e, sl] = rd[e, sl] + rs[e, sl]
            return carry
        lax.fori_loop(0, GCB, ab, 0)

    def drain_write(b):
        pltpu.make_async_copy(rds[b], g_hbm.at[pl.ds(0, GCB)], wsems[b]).wait()

    def body(i, carry):
        k0 = i * _GNB

        @pl.when(i > 0)
        def _():
            for b in range(_GNB):
                drain_write(b)

        descs = [issue(k0 + b, b) for b in range(_GNB)]
        for b in range(_GNB):
            descs[b][0].wait()
            descs[b][1].wait()
            add(b)
            pltpu.async_copy(
                rds[b], g_hbm.at[pl.ds(ebase + (k0 + b) * GCB, GCB)], wsems[b])
        return carry

    lax.fori_loop(0, NGC // _GNB, body, 0)
    for b in range(NGC % _GNB):
        k = (NGC // _GNB) * _GNB + b
        drain_write(b)
        da, db = issue(k, b)
        da.wait()
        db.wait()
        add(b)
        pltpu.async_copy(
            rds[b], g_hbm.at[pl.ds(ebase + k * GCB, GCB)], wsems[b])
    for b in range(_GNB):
        if b >= NGC % _GNB:
            drain_write(b)
    for b in range(NGC % _GNB):
        drain_write(b)


_MB = 2000


def _math_body(g_ref, ea_ref, w2_ref, w16_ref, sb_ref, m_ref):
    eav = jax.nn.softplus(
        jnp.sum(ea_ref[...] * w16_ref[...], axis=1, keepdims=True)
        + sb_ref[0, 0])
    z = g_ref[...] + eav * w2_ref[...]
    m_ref[...] = jax.nn.sigmoid(z[:, :D]) * jax.nn.softplus(z[:, D:])


def _scatter_body(m_hbm, dsc3_hbm, z_hbm, part_hbm,
                  idx2, mb0, mb1, sem0, sem1, agg_sh):
    c = lax.axis_index("c")
    s = lax.axis_index("s")
    wid = c * NS + s
    row0 = s * RPS
    ebase = wid * EPW

    pltpu.sync_copy(z_hbm.at[pl.ds(row0, RPS)], agg_sh.at[pl.ds(row0, RPS)])
    pltpu.sync_copy(dsc3_hbm.at[wid], idx2)
    plsc.subcore_barrier()

    def load(k, mb, sem):
        return pltpu.async_copy(m_hbm.at[pl.ds(ebase + k * SCB, SCB)], mb, sem)

    def body(i, carry):
        k0 = i * 2
        k1 = k0 + 1
        d0 = load(k0, mb0, sem0)
        d1 = load(k1, mb1, sem1)
        d0.wait()
        pltpu.sync_copy(mb0, agg_sh.at[idx2.at[k0]], add=True)
        d1.wait()
        pltpu.sync_copy(mb1, agg_sh.at[idx2.at[k1]], add=True)
        return carry

    lax.fori_loop(0, NSC // 2, body, 0)
    load(NSC - 1, mb0, sem0).wait()
    pltpu.sync_copy(mb0, agg_sh.at[idx2.at[NSC - 1]], add=True)

    plsc.subcore_barrier()
    pltpu.sync_copy(agg_sh.at[pl.ds(row0, RPS)],
                    part_hbm.at[c, pl.ds(row0, RPS)])


def _fin_body(p0_ref, p1_ref, out_ref_in, g_ref, b_ref, y_ref):
    agg = p0_ref[...] + p1_ref[...]
    mean = jnp.mean(agg, axis=0, keepdims=True)
    var = jnp.mean((agg - mean) ** 2, axis=0, keepdims=True)
    bn = (agg - mean) * lax.rsqrt(var + 1e-5) * g_ref[...] + b_ref[...]
    y_ref[...] = 2.0 * out_ref_in[...] + bn


def kernel(h, edge_index, edge_weight, edge_attr, data, lin0_W, lin0_b,
           short_W, short_b, Wf, bf, Ws, bs, bn_gamma, bn_beta):
    w0t = lin0_W.T
    wd = jnp.concatenate([Wf[:, :D].T, Ws[:, :D].T], axis=1)
    bd = jnp.concatenate([bf, bs])[None, :]
    wsrc = jnp.concatenate([Wf[:, D:2 * D].T, Ws[:, D:2 * D].T], axis=1)
    w2 = jnp.concatenate([Wf[:, 2 * D], Ws[:, 2 * D]])[None, :]
    src = edge_index[0]
    dst = edge_index[1]
    b0 = lin0_b[None, :]

    out, td, ts = pl.pallas_call(
        _prep_body,
        grid=(N // _PB,),
        in_specs=[
            pl.BlockSpec((_PB, D), lambda i: (i, 0)),
            pl.BlockSpec((D, D), lambda i: (0, 0)),
            pl.BlockSpec((1, D), lambda i: (0, 0)),
            pl.BlockSpec((D, 2 * D), lambda i: (0, 0)),
            pl.BlockSpec((1, 2 * D), lambda i: (0, 0)),
            pl.BlockSpec((D, 2 * D), lambda i: (0, 0)),
        ],
        out_specs=[
            pl.BlockSpec((_PB, D), lambda i: (i, 0)),
            pl.BlockSpec((_PB, 2 * D), lambda i: (i, 0)),
            pl.BlockSpec((_PB, 2 * D), lambda i: (i, 0)),
        ],
        out_shape=[
            jax.ShapeDtypeStruct((N, D), jnp.float32),
            jax.ShapeDtypeStruct((N, 2 * D), jnp.float32),
            jax.ShapeDtypeStruct((N, 2 * D), jnp.float32),
        ],
    )(h, w0t, b0, wd, bd, wsrc)

    dst3 = dst.reshape(NW, NGC, GCB)
    src3 = src.reshape(NW, NGC, GCB)
    g = pl.kernel(
        _gather_body,
        out_type=jax.ShapeDtypeStruct((E, 2 * D), jnp.float32),
        mesh=plsc.VectorSubcoreMesh(core_axis_name="c", subcore_axis_name="s"),
        scratch_types=[
            pltpu.VMEM((NGC, GCB), jnp.int32),
            pltpu.VMEM((NGC, GCB), jnp.int32),
            [pltpu.VMEM((GCB, 2 * D), jnp.float32)] * _GNB,
            [pltpu.VMEM((GCB, 2 * D), jnp.float32)] * _GNB,
            [pltpu.SemaphoreType.DMA] * _GNB,
            [pltpu.SemaphoreType.DMA] * _GNB,
        ],
    )(td, ts, dst3, src3)

    m = pl.pallas_call(
        _math_body,
        grid=(E // _MB,),
        in_specs=[
            pl.BlockSpec((_MB, 2 * D), lambda i: (i, 0)),
            pl.BlockSpec((_MB, DE), lambda i: (i, 0)),
            pl.BlockSpec((1, 2 * D), lambda i: (0, 0)),
            pl.BlockSpec((1, DE), lambda i: (0, 0)),
            pl.BlockSpec((1, 1), lambda i: (0, 0)),
        ],
        out_specs=pl.BlockSpec((_MB, D), lambda i: (i, 0)),
        out_shape=jax.ShapeDtypeStruct((E, D), jnp.float32),
    )(g, edge_attr, w2, short_W, short_b[None, :])

    zeros = jnp.zeros((NPAD, D), jnp.float32)
    dsc3 = dst.reshape(NW, NSC, SCB)
    parts = pl.kernel(
        _scatter_body,
        out_type=jax.ShapeDtypeStruct((NC, NPAD, D), jnp.float32),
        mesh=plsc.VectorSubcoreMesh(core_axis_name="c", subcore_axis_name="s"),
        scratch_types=[
            pltpu.VMEM((NSC, SCB), jnp.int32),
            pltpu.VMEM((SCB, D), jnp.float32),
            pltpu.VMEM((SCB, D), jnp.float32),
            pltpu.SemaphoreType.DMA,
            pltpu.SemaphoreType.DMA,
            pltpu.VMEM_SHARED((NPAD, D), jnp.float32),
        ],
    )(m, dsc3, zeros)

    y = pl.pallas_call(
        _fin_body,
        out_shape=jax.ShapeDtypeStruct((N, D), jnp.float32),
    )(parts[0, :N], parts[1, :N], out, bn_gamma[None, :], bn_beta[None, :])
    return y
